# baseline (speedup 1.0000x reference)
import jax
import jax.numpy as jnp
from jax import lax
from jax.experimental import pallas as pl
from jax.experimental.pallas import tpu as pltpu
from jax.experimental.pallas import tpu_sc as plsc

_R, _D = 64, 8192
_L = 16
_NCHUNK = _D // _L
_UNROLL = 4
_NC, _NS = 2, 16
_NW = _NC * _NS
_ROWS_PER_W = _R // _NW


def _sparsemax_body(x_hbm, out_hbm, row_v, idx_v, zero_v):
    cid = lax.axis_index("c")
    sid = lax.axis_index("s")
    wid = sid * _NC + cid
    lane = lax.iota(jnp.int32, _L)
    zvec = jnp.zeros((_L,), jnp.float32)

    def z_body(c):
        zero_v[pl.ds(c * _L, _L)] = zvec

    plsc.parallel_loop(0, _NCHUNK, unroll=_UNROLL)(z_body)

    def row_body(r, row_carry):
        row = wid * _ROWS_PER_W + r
        pltpu.sync_copy(x_hbm.at[row], row_v)

        def mx_body(cb, accs):
            return tuple(
                jnp.maximum(accs[j], row_v[pl.ds((cb + j) * _L, _L)])
                for j in range(_UNROLL)
            )

        ninf = jnp.full((_L,), -jnp.inf, jnp.float32)
        accs = plsc.parallel_loop(0, _NCHUNK, step=_UNROLL,
                                  carry=(ninf,) * _UNROLL)(mx_body)
        acc = accs[0]
        for j in range(1, _UNROLL):
            acc = jnp.maximum(acc, accs[j])

        srt, _ = plsc.sort_key_val(acc, acc, descending=True)
        cums = plsc.cumsum(srt)
        rho = (lane + 1).astype(jnp.float32)
        tvec = (cums - 1.0) / rho
        thr_raw = jnp.max(tvec)
        thr = thr_raw - (1e-3 + 1e-3 * jnp.abs(thr_raw))

        def cp_body(c, carry):
            off, ivec = carry
            v = row_v[pl.ds(c * _L, _L)]
            msk = v > thr
            plsc.store_compressed(idx_v.at[pl.ds(off, _L)], ivec, mask=msk)
            cnt = plsc.all_reduce_population_count(msk)
            return off + cnt[0], ivec + _L

        m, _ = plsc.parallel_loop(0, _NCHUNK, unroll=_UNROLL,
                                  carry=(jnp.int32(0), lane))(cp_body)
        nc = (m + _L - 1) // _L

        def newton_body(state):
            tau, _, it = state

            def acc_body(c, carry):
                sv, cv = carry
                valid = c * _L + lane < m
                iv = idx_v[pl.ds(c * _L, _L)]
                v = plsc.load_gather(row_v, [iv], mask=valid)
                sel = jnp.logical_and(valid, v > tau)
                sv = sv + jnp.where(sel, v, 0.0)
                cv = cv + jnp.where(sel, 1.0, 0.0)
                return sv, cv

            sv, cv = lax.fori_loop(0, nc, acc_body, (zvec, zvec))
            s = jnp.sum(sv)
            k = jnp.maximum(jnp.sum(cv), 1.0)
            tau_v = jnp.broadcast_to(s - 1.0, (_L,)) / jnp.broadcast_to(k, (_L,))
            return tau_v[0], tau, it + 1

        def newton_cond(state):
            tau, prev, it = state
            return jnp.logical_and(tau != prev, it < 64)

        tau, _, _ = lax.while_loop(newton_cond, newton_body,
                                   (thr, thr - 1.0, jnp.int32(0)))

        def sc_body(c, carry):
            valid = c * _L + lane < m
            iv = idx_v[pl.ds(c * _L, _L)]
            v = plsc.load_gather(row_v, [iv], mask=valid)
            sel = jnp.logical_and(valid, v > tau)
            plsc.store_scatter(zero_v, [iv], v - tau, mask=sel)
            return carry

        lax.fori_loop(0, nc, sc_body, jnp.int32(0))
        pltpu.sync_copy(zero_v, out_hbm.at[row])

        def us_body(c, carry):
            valid = c * _L + lane < m
            iv = idx_v[pl.ds(c * _L, _L)]
            v = plsc.load_gather(row_v, [iv], mask=valid)
            sel = jnp.logical_and(valid, v > tau)
            plsc.store_scatter(zero_v, [iv], zvec, mask=sel)
            return carry

        lax.fori_loop(0, nc, us_body, jnp.int32(0))
        return row_carry

    lax.fori_loop(0, _ROWS_PER_W, row_body, jnp.int32(0))


def _make_call():
    return pl.kernel(
        _sparsemax_body,
        out_type=jax.ShapeDtypeStruct((_R, _D), jnp.float32),
        mesh=plsc.VectorSubcoreMesh(core_axis_name="c", subcore_axis_name="s",
                                    num_cores=_NC, num_subcores=_NS),
        scratch_types=[
            pltpu.VMEM((_D,), jnp.float32),
            pltpu.VMEM((_D,), jnp.int32),
            pltpu.VMEM((_D,), jnp.float32),
        ],
        compiler_params=pltpu.CompilerParams(needs_layout_passes=False,
                                             skip_device_barrier=True),
    )


@jax.jit
def kernel(input):
    return _make_call()(input)


# speedup vs baseline: 1.0253x; 1.0253x over previous
import jax
import jax.numpy as jnp
from jax import lax
from jax.experimental import pallas as pl
from jax.experimental.pallas import tpu as pltpu
from jax.experimental.pallas import tpu_sc as plsc

_R, _D = 64, 8192
_L = 16
_NCHUNK = _D // _L
_UNROLL = 8
_NC, _NS = 2, 16
_NW = _NC * _NS
_ROWS_PER_W = _R // _NW


def _sparsemax_body(x_hbm, out_hbm, row_v, idx_v, zero_v):
    cid = lax.axis_index("c")
    sid = lax.axis_index("s")
    wid = sid * _NC + cid
    lane = lax.iota(jnp.int32, _L)
    zvec = jnp.zeros((_L,), jnp.float32)

    def z_body(c):
        zero_v[pl.ds(c * _L, _L)] = zvec

    plsc.parallel_loop(0, _NCHUNK, unroll=_UNROLL)(z_body)

    def row_body(r, row_carry):
        row = wid * _ROWS_PER_W + r
        pltpu.sync_copy(x_hbm.at[row], row_v)

        def mx_body(c, acc):
            return jnp.maximum(acc, row_v[pl.ds(c * _L, _L)])

        ninf = jnp.full((_L,), -jnp.inf, jnp.float32)
        acc = plsc.parallel_loop(0, 32, unroll=4, carry=ninf)(mx_body)

        srt, _ = plsc.sort_key_val(acc, acc, descending=True)
        cums = plsc.cumsum(srt)
        rho = (lane + 1).astype(jnp.float32)
        tvec = (cums - 1.0) / rho
        thr_raw = jnp.max(tvec)
        thr = thr_raw - (1e-3 + 1e-3 * jnp.abs(thr_raw))

        def cp_body(c, carry):
            off, ivec = carry
            v = row_v[pl.ds(c * _L, _L)]
            msk = v > thr
            plsc.store_compressed(idx_v.at[pl.ds(off, _L)], ivec, mask=msk)
            cnt = plsc.all_reduce_population_count(msk)
            return off + cnt[0], ivec + _L

        m, _ = plsc.parallel_loop(0, _NCHUNK, unroll=_UNROLL,
                                  carry=(jnp.int32(0), lane))(cp_body)
        nc = (m + _L - 1) // _L

        def newton_body(state):
            tau, _, it = state

            def acc_body(c, carry):
                sv, cv = carry
                valid = c * _L + lane < m
                iv = idx_v[pl.ds(c * _L, _L)]
                v = plsc.load_gather(row_v, [iv], mask=valid)
                sel = jnp.logical_and(valid, v > tau)
                sv = sv + jnp.where(sel, v, 0.0)
                cv = cv + jnp.where(sel, 1.0, 0.0)
                return sv, cv

            sv, cv = lax.fori_loop(0, nc, acc_body, (zvec, zvec))
            s = jnp.sum(sv)
            k = jnp.maximum(jnp.sum(cv), 1.0)
            tau_v = jnp.broadcast_to(s - 1.0, (_L,)) / jnp.broadcast_to(k, (_L,))
            return tau_v[0], tau, it + 1

        def newton_cond(state):
            tau, prev, it = state
            return jnp.logical_and(tau != prev, it < 64)

        tau, _, _ = lax.while_loop(newton_cond, newton_body,
                                   (thr, thr - 1.0, jnp.int32(0)))

        def sc_body(c, carry):
            valid = c * _L + lane < m
            iv = idx_v[pl.ds(c * _L, _L)]
            v = plsc.load_gather(row_v, [iv], mask=valid)
            sel = jnp.logical_and(valid, v > tau)
            plsc.store_scatter(zero_v, [iv], v - tau, mask=sel)
            return carry

        lax.fori_loop(0, nc, sc_body, jnp.int32(0))
        pltpu.sync_copy(zero_v, out_hbm.at[row])

        def us_body(c, carry):
            valid = c * _L + lane < m
            iv = idx_v[pl.ds(c * _L, _L)]
            v = plsc.load_gather(row_v, [iv], mask=valid)
            sel = jnp.logical_and(valid, v > tau)
            plsc.store_scatter(zero_v, [iv], zvec, mask=sel)
            return carry

        lax.fori_loop(0, nc, us_body, jnp.int32(0))
        return row_carry

    lax.fori_loop(0, _ROWS_PER_W, row_body, jnp.int32(0))


def _make_call():
    return pl.kernel(
        _sparsemax_body,
        out_type=jax.ShapeDtypeStruct((_R, _D), jnp.float32),
        mesh=plsc.VectorSubcoreMesh(core_axis_name="c", subcore_axis_name="s",
                                    num_cores=_NC, num_subcores=_NS),
        scratch_types=[
            pltpu.VMEM((_D,), jnp.float32),
            pltpu.VMEM((_D,), jnp.int32),
            pltpu.VMEM((_D,), jnp.float32),
        ],
        compiler_params=pltpu.CompilerParams(needs_layout_passes=False,
                                             skip_device_barrier=True),
    )


@jax.jit
def kernel(input):
    return _make_call()(input)


# speedup vs baseline: 1.0377x; 1.0121x over previous
import jax
import jax.numpy as jnp
from jax import lax
from jax.experimental import pallas as pl
from jax.experimental.pallas import tpu as pltpu
from jax.experimental.pallas import tpu_sc as plsc

_R, _D = 64, 8192
_L = 16
_NCHUNK = _D // _L
_UNROLL = 8
_NC, _NS = 2, 16
_NW = _NC * _NS
_ROWS_PER_W = _R // _NW


def _sparsemax_body(x_hbm, out_hbm, row_v, idx_v, zero_v, in_sem, out_sem):
    cid = lax.axis_index("c")
    sid = lax.axis_index("s")
    wid = sid * _NC + cid
    lane = lax.iota(jnp.int32, _L)
    zvec = jnp.zeros((_L,), jnp.float32)

    pltpu.async_copy(x_hbm.at[wid * _ROWS_PER_W], row_v, in_sem)

    def z_body(c):
        zero_v[pl.ds(c * _L, _L)] = zvec

    plsc.parallel_loop(0, _NCHUNK, unroll=_UNROLL)(z_body)

    def row_body(r, row_carry):
        row = wid * _ROWS_PER_W + r
        pltpu.make_async_copy(x_hbm.at[row], row_v, in_sem).wait()

        def mx_body(cb, accs):
            return tuple(
                jnp.maximum(accs[j], row_v[pl.ds((cb + j) * _L, _L)])
                for j in range(_UNROLL)
            )

        ninf = jnp.full((_L,), -jnp.inf, jnp.float32)
        accs = plsc.parallel_loop(0, _NCHUNK, step=_UNROLL,
                                  carry=(ninf,) * _UNROLL)(mx_body)
        acc = accs[0]
        for j in range(1, _UNROLL):
            acc = jnp.maximum(acc, accs[j])

        srt, _ = plsc.sort_key_val(acc, acc, descending=True)
        cums = plsc.cumsum(srt)
        rho = (lane + 1).astype(jnp.float32)
        tvec = (cums - 1.0) / rho
        thr_raw = jnp.max(tvec)
        thr = thr_raw - (1e-3 + 1e-3 * jnp.abs(thr_raw))

        def cp_body(c, carry):
            off, ivec = carry
            v = row_v[pl.ds(c * _L, _L)]
            msk = v > thr
            plsc.store_compressed(idx_v.at[pl.ds(off, _L)], ivec, mask=msk)
            cnt = plsc.all_reduce_population_count(msk)
            return off + cnt[0], ivec + _L

        m, _ = plsc.parallel_loop(0, _NCHUNK, unroll=_UNROLL,
                                  carry=(jnp.int32(0), lane))(cp_body)
        nc = (m + _L - 1) // _L

        def newton_body(state):
            tau, _, it = state

            def acc_body(c, carry):
                sv, cv = carry
                valid = c * _L + lane < m
                iv = idx_v[pl.ds(c * _L, _L)]
                v = plsc.load_gather(row_v, [iv], mask=valid)
                sel = jnp.logical_and(valid, v > tau)
                sv = sv + jnp.where(sel, v, 0.0)
                cv = cv + jnp.where(sel, 1.0, 0.0)
                return sv, cv

            sv, cv = lax.fori_loop(0, nc, acc_body, (zvec, zvec))
            s = jnp.sum(sv)
            k = jnp.maximum(jnp.sum(cv), 1.0)
            tau_v = jnp.broadcast_to(s - 1.0, (_L,)) / jnp.broadcast_to(k, (_L,))
            return tau_v[0], tau, it + 1

        def newton_cond(state):
            tau, prev, it = state
            return jnp.logical_and(tau != prev, it < 64)

        tau, _, _ = lax.while_loop(newton_cond, newton_body,
                                   (thr, thr - 1.0, jnp.int32(0)))

        def sc_body(c, carry):
            valid = c * _L + lane < m
            iv = idx_v[pl.ds(c * _L, _L)]
            v = plsc.load_gather(row_v, [iv], mask=valid)
            sel = jnp.logical_and(valid, v > tau)
            plsc.store_scatter(zero_v, [iv], v - tau, mask=sel)
            return carry

        lax.fori_loop(0, nc, sc_body, jnp.int32(0))

        pltpu.async_copy(zero_v, out_hbm.at[row], out_sem)
        nxt = wid * _ROWS_PER_W + jnp.minimum(r + 1, _ROWS_PER_W - 1)
        pltpu.async_copy(x_hbm.at[nxt], row_v, in_sem)
        pltpu.make_async_copy(zero_v, out_hbm.at[row], out_sem).wait()

        def us_body(c, carry):
            valid = c * _L + lane < m
            iv = idx_v[pl.ds(c * _L, _L)]
            plsc.store_scatter(zero_v, [iv], zvec, mask=valid)
            return carry

        lax.fori_loop(0, nc, us_body, jnp.int32(0))
        return row_carry

    lax.fori_loop(0, _ROWS_PER_W, row_body, jnp.int32(0))
    pltpu.make_async_copy(x_hbm.at[wid * _ROWS_PER_W], row_v, in_sem).wait()


def _make_call():
    return pl.kernel(
        _sparsemax_body,
        out_type=jax.ShapeDtypeStruct((_R, _D), jnp.float32),
        mesh=plsc.VectorSubcoreMesh(core_axis_name="c", subcore_axis_name="s",
                                    num_cores=_NC, num_subcores=_NS),
        scratch_types=[
            pltpu.VMEM((_D,), jnp.float32),
            pltpu.VMEM((_D,), jnp.int32),
            pltpu.VMEM((_D,), jnp.float32),
            pltpu.SemaphoreType.DMA,
            pltpu.SemaphoreType.DMA,
        ],
        compiler_params=pltpu.CompilerParams(needs_layout_passes=False,
                                             skip_device_barrier=True),
    )


@jax.jit
def kernel(input):
    return _make_call()(input)


# speedup vs baseline: 1.0694x; 1.0306x over previous
import jax
import jax.numpy as jnp
from jax import lax
from jax.experimental import pallas as pl
from jax.experimental.pallas import tpu as pltpu
from jax.experimental.pallas import tpu_sc as plsc

_R, _D = 64, 8192
_L = 16
_NCHUNK = _D // _L
_UNROLL = 8
_NC, _NS = 2, 16
_NW = _NC * _NS
_ROWS_PER_W = _R // _NW


def _sparsemax_body(x_hbm, out_hbm, row_v, idx_v, zero_v, in_sem, out_sem):
    cid = lax.axis_index("c")
    sid = lax.axis_index("s")
    wid = sid * _NC + cid
    lane = lax.iota(jnp.int32, _L)
    zvec = jnp.zeros((_L,), jnp.float32)

    pltpu.async_copy(x_hbm.at[wid * _ROWS_PER_W], row_v, in_sem)

    def z_body(c):
        zero_v[pl.ds(c * _L, _L)] = zvec

    plsc.parallel_loop(0, _NCHUNK, unroll=_UNROLL)(z_body)

    def row_body(r, row_carry):
        row = wid * _ROWS_PER_W + r
        pltpu.make_async_copy(x_hbm.at[row], row_v, in_sem).wait()

        def mx_body(cb, accs):
            return tuple(
                jnp.maximum(accs[j], row_v[pl.ds((cb + j) * _L, _L)])
                for j in range(_UNROLL)
            )

        ninf = jnp.full((_L,), -jnp.inf, jnp.float32)
        accs = plsc.parallel_loop(0, _NCHUNK, step=_UNROLL,
                                  carry=(ninf,) * _UNROLL)(mx_body)
        acc = accs[0]
        for j in range(1, _UNROLL):
            acc = jnp.maximum(acc, accs[j])

        srt, _ = plsc.sort_key_val(acc, acc, descending=True)
        cums = plsc.cumsum(srt)
        rho = (lane + 1).astype(jnp.float32)
        tvec = (cums - 1.0) / rho
        thr_raw = jnp.max(tvec)
        thr = thr_raw - (1e-3 + 1e-3 * jnp.abs(thr_raw))

        def cp_body(c, carry):
            off, ivec = carry
            v = row_v[pl.ds(c * _L, _L)]
            msk = v > thr
            plsc.store_compressed(idx_v.at[pl.ds(off, _L)], ivec, mask=msk)
            cnt = plsc.all_reduce_population_count(msk)
            return off + cnt[0], ivec + _L

        m, _ = plsc.parallel_loop(0, _NCHUNK, unroll=_UNROLL,
                                  carry=(jnp.int32(0), lane))(cp_body)
        nc = (m + _L - 1) // _L

        def newton_body(state):
            tau, _, it = state

            def acc_body(c, carry):
                sv, cv = carry
                valid = c * _L + lane < m
                iv = idx_v[pl.ds(c * _L, _L)]
                v = plsc.load_gather(row_v, [iv], mask=valid)
                sel = jnp.logical_and(valid, v > tau)
                sv = sv + jnp.where(sel, v, 0.0)
                cv = cv + jnp.where(sel, 1.0, 0.0)
                return sv, cv

            sv, cv = lax.fori_loop(0, nc, acc_body, (zvec, zvec))
            s = jnp.sum(sv)
            k = jnp.maximum(jnp.sum(cv), 1.0)
            tau_v = jnp.broadcast_to(s - 1.0, (_L,)) / jnp.broadcast_to(k, (_L,))
            return tau_v[0], tau, it + 1

        def newton_cond(state):
            tau, prev, it = state
            return jnp.logical_and(tau != prev, it < 64)

        tau, _, _ = lax.while_loop(newton_cond, newton_body,
                                   (thr, thr - 1.0, jnp.int32(0)))

        def sc_body(c, carry):
            valid = c * _L + lane < m
            iv = idx_v[pl.ds(c * _L, _L)]
            v = plsc.load_gather(row_v, [iv], mask=valid)
            sel = jnp.logical_and(valid, v > tau)
            plsc.store_scatter(zero_v, [iv], v - tau, mask=sel)
            return carry

        lax.fori_loop(0, nc, sc_body, jnp.int32(0))

        pltpu.async_copy(zero_v, out_hbm.at[row], out_sem)

        @pl.when(r + 1 < _ROWS_PER_W)
        def _():
            pltpu.async_copy(x_hbm.at[row + 1], row_v, in_sem)

        pltpu.make_async_copy(zero_v, out_hbm.at[row], out_sem).wait()

        def us_body(c, carry):
            valid = c * _L + lane < m
            iv = idx_v[pl.ds(c * _L, _L)]
            plsc.store_scatter(zero_v, [iv], zvec, mask=valid)
            return carry

        lax.fori_loop(0, nc, us_body, jnp.int32(0))
        return row_carry

    lax.fori_loop(0, _ROWS_PER_W, row_body, jnp.int32(0))


def _make_call():
    return pl.kernel(
        _sparsemax_body,
        out_type=jax.ShapeDtypeStruct((_R, _D), jnp.float32),
        mesh=plsc.VectorSubcoreMesh(core_axis_name="c", subcore_axis_name="s",
                                    num_cores=_NC, num_subcores=_NS),
        scratch_types=[
            pltpu.VMEM((_D,), jnp.float32),
            pltpu.VMEM((_D,), jnp.int32),
            pltpu.VMEM((_D,), jnp.float32),
            pltpu.SemaphoreType.DMA,
            pltpu.SemaphoreType.DMA,
        ],
        compiler_params=pltpu.CompilerParams(needs_layout_passes=False,
                                             skip_device_barrier=True),
    )


@jax.jit
def kernel(input):
    return _make_call()(input)


# speedup vs baseline: 1.0707x; 1.0012x over previous
import jax
import jax.numpy as jnp
from jax import lax
from jax.experimental import pallas as pl
from jax.experimental.pallas import tpu as pltpu
from jax.experimental.pallas import tpu_sc as plsc

_R, _D = 64, 8192
_L = 16
_NCHUNK = _D // _L
_UNROLL = 8
_NC, _NS = 2, 16
_NW = _NC * _NS
_ROWS_PER_W = _R // _NW


def _sparsemax_body(x_hbm, out_hbm, row_v, idx_v, zero_v, in_sem, out_sem):
    cid = lax.axis_index("c")
    sid = lax.axis_index("s")
    wid = sid * _NC + cid
    lane = lax.iota(jnp.int32, _L)
    zvec = jnp.zeros((_L,), jnp.float32)

    pltpu.async_copy(x_hbm.at[wid * _ROWS_PER_W], row_v, in_sem)

    def z_body(c):
        zero_v[pl.ds(c * _L, _L)] = zvec

    plsc.parallel_loop(0, _NCHUNK, unroll=_UNROLL)(z_body)

    def row_body(r, row_carry):
        row = wid * _ROWS_PER_W + r
        pltpu.make_async_copy(x_hbm.at[row], row_v, in_sem).wait()

        def mx_body(cb, accs):
            return tuple(
                jnp.maximum(accs[j], row_v[pl.ds((cb + j) * _L, _L)])
                for j in range(_UNROLL)
            )

        ninf = jnp.full((_L,), -jnp.inf, jnp.float32)
        accs = plsc.parallel_loop(0, 128, step=_UNROLL,
                                  carry=(ninf,) * _UNROLL)(mx_body)
        acc = accs[0]
        for j in range(1, _UNROLL):
            acc = jnp.maximum(acc, accs[j])

        srt, _ = plsc.sort_key_val(acc, acc, descending=True)
        cums = plsc.cumsum(srt)
        rho = (lane + 1).astype(jnp.float32)
        tvec = (cums - 1.0) / rho
        thr_raw = jnp.max(tvec)
        thr = thr_raw - (1e-3 + 1e-3 * jnp.abs(thr_raw))

        def cp_body(c, carry):
            off, ivec = carry
            v = row_v[pl.ds(c * _L, _L)]
            msk = v > thr
            plsc.store_compressed(idx_v.at[pl.ds(off, _L)], ivec, mask=msk)
            cnt = plsc.all_reduce_population_count(msk)
            return off + cnt[0], ivec + _L

        m, _ = plsc.parallel_loop(0, _NCHUNK, unroll=_UNROLL,
                                  carry=(jnp.int32(0), lane))(cp_body)
        nc = (m + _L - 1) // _L

        def newton_body(state):
            tau, _, it = state

            def acc_body(c, carry):
                sv, cv = carry
                valid = c * _L + lane < m
                iv = idx_v[pl.ds(c * _L, _L)]
                v = plsc.load_gather(row_v, [iv], mask=valid)
                sel = jnp.logical_and(valid, v > tau)
                sv = sv + jnp.where(sel, v, 0.0)
                cv = cv + jnp.where(sel, 1.0, 0.0)
                return sv, cv

            sv, cv = lax.fori_loop(0, nc, acc_body, (zvec, zvec))
            s = jnp.sum(sv)
            k = jnp.maximum(jnp.sum(cv), 1.0)
            tau_v = jnp.broadcast_to(s - 1.0, (_L,)) / jnp.broadcast_to(k, (_L,))
            return tau_v[0], tau, it + 1

        def newton_cond(state):
            tau, prev, it = state
            return jnp.logical_and(tau != prev, it < 64)

        tau, _, _ = lax.while_loop(newton_cond, newton_body,
                                   (thr, thr - 1.0, jnp.int32(0)))

        def sc_body(c, carry):
            valid = c * _L + lane < m
            iv = idx_v[pl.ds(c * _L, _L)]
            v = plsc.load_gather(row_v, [iv], mask=valid)
            sel = jnp.logical_and(valid, v > tau)
            plsc.store_scatter(zero_v, [iv], v - tau, mask=sel)
            return carry

        lax.fori_loop(0, nc, sc_body, jnp.int32(0))

        pltpu.async_copy(zero_v, out_hbm.at[row], out_sem)

        @pl.when(r + 1 < _ROWS_PER_W)
        def _():
            pltpu.async_copy(x_hbm.at[row + 1], row_v, in_sem)

        pltpu.make_async_copy(zero_v, out_hbm.at[row], out_sem).wait()

        def us_body(c, carry):
            valid = c * _L + lane < m
            iv = idx_v[pl.ds(c * _L, _L)]
            plsc.store_scatter(zero_v, [iv], zvec, mask=valid)
            return carry

        lax.fori_loop(0, nc, us_body, jnp.int32(0))
        return row_carry

    lax.fori_loop(0, _ROWS_PER_W, row_body, jnp.int32(0))


def _make_call():
    return pl.kernel(
        _sparsemax_body,
        out_type=jax.ShapeDtypeStruct((_R, _D), jnp.float32),
        mesh=plsc.VectorSubcoreMesh(core_axis_name="c", subcore_axis_name="s",
                                    num_cores=_NC, num_subcores=_NS),
        scratch_types=[
            pltpu.VMEM((_D,), jnp.float32),
            pltpu.VMEM((_D,), jnp.int32),
            pltpu.VMEM((_D,), jnp.float32),
            pltpu.SemaphoreType.DMA,
            pltpu.SemaphoreType.DMA,
        ],
        compiler_params=pltpu.CompilerParams(needs_layout_passes=False,
                                             skip_device_barrier=True),
    )


@jax.jit
def kernel(input):
    return _make_call()(input)
